# baseline (speedup 1.0000x reference)
import functools

import jax
import jax.numpy as jnp
from jax import lax
from jax.experimental import pallas as pl
from jax.experimental.pallas import tpu as pltpu
from jax.experimental.pallas import tpu_sc as plsc

NC = 2
NS = 16
NW = NC * NS
CHUNK = 128


def _gather_body(bpw, uid_h, iid_h, p_h, q_h, nf_h, outp_h, outq_h, outn_h,
                 uid_v, iid_v, pv, qv, nv, sem):
  wid = lax.axis_index("s") * NC + lax.axis_index("c")
  base = wid * bpw
  pltpu.sync_copy(uid_h.at[pl.ds(base, bpw)], uid_v)
  pltpu.sync_copy(iid_h.at[pl.ds(base, bpw)], iid_v)
  handles = []
  for j in range(bpw // CHUNK):
    sl = pl.ds(j * CHUNK, CHUNK)
    handles.append(pltpu.async_copy(p_h.at[uid_v.at[sl]], pv.at[sl], sem))
    handles.append(pltpu.async_copy(q_h.at[iid_v.at[sl]], qv.at[sl], sem))
    handles.append(pltpu.async_copy(nf_h.at[iid_v.at[sl]], nv.at[sl], sem))
  for h in handles:
    h.wait()
  pltpu.sync_copy(pv, outp_h.at[pl.ds(base, bpw)])
  pltpu.sync_copy(qv, outq_h.at[pl.ds(base, bpw)])
  pltpu.sync_copy(nv, outn_h.at[pl.ds(base, bpw)])


def _sc_gather(uid, iid, P, Q, nf):
  b = uid.shape[0]
  bpw = b // NW
  dp, dq, dn = P.shape[1], Q.shape[1], nf.shape[1]
  mesh = plsc.VectorSubcoreMesh(core_axis_name="c", subcore_axis_name="s")
  fn = pl.kernel(
      functools.partial(_gather_body, bpw),
      out_type=(
          jax.ShapeDtypeStruct((b, dp), jnp.float32),
          jax.ShapeDtypeStruct((b, dq), jnp.float32),
          jax.ShapeDtypeStruct((b, dn), jnp.float32),
      ),
      mesh=mesh,
      scratch_types=(
          pltpu.VMEM((bpw,), jnp.int32),
          pltpu.VMEM((bpw,), jnp.int32),
          pltpu.VMEM((bpw, dp), jnp.float32),
          pltpu.VMEM((bpw, dq), jnp.float32),
          pltpu.VMEM((bpw, dn), jnp.float32),
          pltpu.SemaphoreType.DMA,
      ),
      compiler_params=pltpu.CompilerParams(use_tc_tiling_on_sc=False),
  )
  return fn(uid, iid, P, Q, nf)


def _score_body(pg, qg, ng, weu, w, wei, out):
  pu = jnp.dot(pg[...], weu[...], preferred_element_type=jnp.float32)
  t = jnp.dot(ng[...], w[...], preferred_element_type=jnp.float32)
  pi = jnp.dot(qg[...], wei[0:32, :], preferred_element_type=jnp.float32)
  pi = pi + jnp.dot(t, wei[32:64, :], preferred_element_type=jnp.float32)
  out[...] = jnp.sum(pu * pi, axis=1)


def _tc_score(pg, qg, ng, weu, w, wei):
  b = pg.shape[0]
  r = 2048
  dp, dq, dn = pg.shape[1], qg.shape[1], ng.shape[1]
  dm = weu.shape[1]
  return pl.pallas_call(
      _score_body,
      grid=(b // r,),
      in_specs=[
          pl.BlockSpec((r, dp), lambda i: (i, 0)),
          pl.BlockSpec((r, dq), lambda i: (i, 0)),
          pl.BlockSpec((r, dn), lambda i: (i, 0)),
          pl.BlockSpec((dp, dm), lambda i: (0, 0)),
          pl.BlockSpec((dn, dm), lambda i: (0, 0)),
          pl.BlockSpec((2 * dm, dm), lambda i: (0, 0)),
      ],
      out_specs=pl.BlockSpec((r,), lambda i: (i,)),
      out_shape=jax.ShapeDtypeStruct((b,), jnp.float32),
  )(pg, qg, ng, weu, w, wei)


@jax.jit
def kernel(uid, iid, P, Q, new_feature, W, weu, wei):
  pg, qg, ng = _sc_gather(uid, iid, P, Q, new_feature)
  return _tc_score(pg, qg, ng, weu, W, wei)


# speedup vs baseline: 1.4993x; 1.4993x over previous
import functools

import jax
import jax.numpy as jnp
from jax import lax
from jax.experimental import pallas as pl
from jax.experimental.pallas import tpu as pltpu
from jax.experimental.pallas import tpu_sc as plsc

NC = 2
NS = 16
NW = NC * NS


def _gather_body(bpw, chunk, uid_h, iid_h, p_h, q_h, nf_h,
                 outp_h, outq_h, outn_h, uid_v, iid_v,
                 pv, qv, nv, semi, semp, semq, semn):
  wid = lax.axis_index("s") * NC + lax.axis_index("c")
  base = wid * bpw
  cu = pltpu.async_copy(uid_h.at[pl.ds(base, bpw)], uid_v, semi)
  ci = pltpu.async_copy(iid_h.at[pl.ds(base, bpw)], iid_v, semi)
  cu.wait()
  ci.wait()

  for h in range(bpw // chunk):
    off = h * chunk

    def issue(j, carry):
      uvec = uid_v[pl.ds(off + j * 16, 16)]
      ivec = iid_v[pl.ds(off + j * 16, 16)]
      for k in range(16):
        r = uvec[k]
        s = ivec[k]
        pltpu.async_copy(p_h.at[r], pv.at[j * 16 + k], semp)
        pltpu.async_copy(q_h.at[s], qv.at[j * 16 + k], semq)
        pltpu.async_copy(nf_h.at[s], nv.at[j * 16 + k], semn)
      return carry

    lax.fori_loop(0, chunk // 16, issue, 0)
    pltpu.make_async_copy(outp_h.at[pl.ds(0, chunk)], pv, semp).wait()
    pltpu.make_async_copy(outq_h.at[pl.ds(0, chunk)], qv, semq).wait()
    pltpu.make_async_copy(outn_h.at[pl.ds(0, chunk)], nv, semn).wait()
    pltpu.sync_copy(pv, outp_h.at[pl.ds(base + off, chunk)])
    pltpu.sync_copy(qv, outq_h.at[pl.ds(base + off, chunk)])
    pltpu.sync_copy(nv, outn_h.at[pl.ds(base + off, chunk)])


def _sc_gather(uid, iid, P, Q, nf):
  b = uid.shape[0]
  bpw = b // NW
  chunk = 256
  dp, dq, dn = P.shape[1], Q.shape[1], nf.shape[1]
  mesh = plsc.VectorSubcoreMesh(core_axis_name="c", subcore_axis_name="s")
  fn = pl.kernel(
      functools.partial(_gather_body, bpw, chunk),
      out_type=(
          jax.ShapeDtypeStruct((b, dp), jnp.float32),
          jax.ShapeDtypeStruct((b, dq), jnp.float32),
          jax.ShapeDtypeStruct((b, dn), jnp.float32),
      ),
      mesh=mesh,
      scratch_types=(
          pltpu.VMEM((bpw,), jnp.int32),
          pltpu.VMEM((bpw,), jnp.int32),
          pltpu.VMEM((chunk, dp), jnp.float32),
          pltpu.VMEM((chunk, dq), jnp.float32),
          pltpu.VMEM((chunk, dn), jnp.float32),
          pltpu.SemaphoreType.DMA,
          pltpu.SemaphoreType.DMA,
          pltpu.SemaphoreType.DMA,
          pltpu.SemaphoreType.DMA,
      ),
  )
  return fn(uid, iid, P, Q, nf)


def _score_body(pg, qg, ng, weu, w, wei, out):
  dm = weu.shape[1]
  pu = jnp.dot(pg[...], weu[...], preferred_element_type=jnp.float32)
  t = jnp.dot(ng[...], w[...], preferred_element_type=jnp.float32)
  pi = jnp.dot(qg[...], wei[0:dm, :], preferred_element_type=jnp.float32)
  pi = pi + jnp.dot(t, wei[dm:2 * dm, :], preferred_element_type=jnp.float32)
  out[...] = jnp.sum(pu * pi, axis=1)


def _tc_score(pg, qg, ng, weu, w, wei):
  b = pg.shape[0]
  r = 2048
  dp, dq, dn = pg.shape[1], qg.shape[1], ng.shape[1]
  dm = weu.shape[1]
  return pl.pallas_call(
      _score_body,
      grid=(b // r,),
      in_specs=[
          pl.BlockSpec((r, dp), lambda i: (i, 0)),
          pl.BlockSpec((r, dq), lambda i: (i, 0)),
          pl.BlockSpec((r, dn), lambda i: (i, 0)),
          pl.BlockSpec((dp, dm), lambda i: (0, 0)),
          pl.BlockSpec((dn, dm), lambda i: (0, 0)),
          pl.BlockSpec((2 * dm, dm), lambda i: (0, 0)),
      ],
      out_specs=pl.BlockSpec((r,), lambda i: (i,)),
      out_shape=jax.ShapeDtypeStruct((b,), jnp.float32),
  )(pg, qg, ng, weu, w, wei)


@jax.jit
def kernel(uid, iid, P, Q, new_feature, W, weu, wei):
  pg, qg, ng = _sc_gather(uid, iid, P, Q, new_feature)
  return _tc_score(pg, qg, ng, weu, W, wei)


# speedup vs baseline: 2.5069x; 1.6720x over previous
import functools

import jax
import jax.numpy as jnp
from jax import lax
from jax.experimental import pallas as pl
from jax.experimental.pallas import tpu as pltpu
from jax.experimental.pallas import tpu_sc as plsc

NC = 2
NS = 16
NW = NC * NS


def _s_body(qt, nt, w, wei, out):
  dm = wei.shape[1]
  a = wei[0:dm, :]
  b2 = jnp.dot(w[...], wei[dm:2 * dm, :], preferred_element_type=jnp.float32)
  s = jnp.einsum("kb,kd->bd", qt[...], a,
                 preferred_element_type=jnp.float32)
  s = s + jnp.einsum("kb,kd->bd", nt[...], b2,
                     preferred_element_type=jnp.float32)
  out[...] = s


def _make_s(qt, nt, w, wei):
  isz = qt.shape[1]
  dq, dn = qt.shape[0], nt.shape[0]
  dm = wei.shape[1]
  blk = 8192
  grid = (isz + blk - 1) // blk
  return pl.pallas_call(
      _s_body,
      grid=(grid,),
      in_specs=[
          pl.BlockSpec((dq, blk), lambda i: (0, i)),
          pl.BlockSpec((dn, blk), lambda i: (0, i)),
          pl.BlockSpec((dn, dm), lambda i: (0, 0)),
          pl.BlockSpec((2 * dm, dm), lambda i: (0, 0)),
      ],
      out_specs=pl.BlockSpec((blk, dm), lambda i: (i, 0)),
      out_shape=jax.ShapeDtypeStruct((isz, dm), jnp.float32),
  )(qt, nt, w, wei)


def _pw_body(pt, weu, out):
  out[...] = jnp.einsum("kb,kd->bd", pt[...], weu[...],
                        preferred_element_type=jnp.float32)


def _make_pw(pt, weu):
  usz = pt.shape[1]
  dp = pt.shape[0]
  dm = weu.shape[1]
  blk = 8192
  grid = (usz + blk - 1) // blk
  return pl.pallas_call(
      _pw_body,
      grid=(grid,),
      in_specs=[
          pl.BlockSpec((dp, blk), lambda i: (0, i)),
          pl.BlockSpec((dp, dm), lambda i: (0, 0)),
      ],
      out_specs=pl.BlockSpec((blk, dm), lambda i: (i, 0)),
      out_shape=jax.ShapeDtypeStruct((usz, dm), jnp.float32),
  )(pt, weu)


def _gather_body(bpw, chunk, uid_h, iid_h, pw_h, s_h, outu_h, outi_h,
                 uid_v, iid_v, uv, iv, semi, semu, sems):
  wid = lax.axis_index("s") * NC + lax.axis_index("c")
  base = wid * bpw
  cu = pltpu.async_copy(uid_h.at[pl.ds(base, bpw)], uid_v, semi)
  ci = pltpu.async_copy(iid_h.at[pl.ds(base, bpw)], iid_v, semi)
  cu.wait()
  ci.wait()

  for h in range(bpw // chunk):
    off = h * chunk

    def issue(j, carry):
      uvec = uid_v[pl.ds(off + j * 16, 16)]
      ivec = iid_v[pl.ds(off + j * 16, 16)]
      for k in range(16):
        pltpu.async_copy(pw_h.at[uvec[k]], uv.at[j * 16 + k], semu)
        pltpu.async_copy(s_h.at[ivec[k]], iv.at[j * 16 + k], sems)
      return carry

    lax.fori_loop(0, chunk // 16, issue, 0)
    pltpu.make_async_copy(outu_h.at[pl.ds(0, chunk)], uv, semu).wait()
    pltpu.make_async_copy(outi_h.at[pl.ds(0, chunk)], iv, sems).wait()
    pltpu.sync_copy(uv, outu_h.at[pl.ds(base + off, chunk)])
    pltpu.sync_copy(iv, outi_h.at[pl.ds(base + off, chunk)])


def _sc_gather(uid, iid, pw, s):
  b = uid.shape[0]
  bpw = b // NW
  chunk = 256
  dm = pw.shape[1]
  mesh = plsc.VectorSubcoreMesh(core_axis_name="c", subcore_axis_name="s")
  fn = pl.kernel(
      functools.partial(_gather_body, bpw, chunk),
      out_type=(
          jax.ShapeDtypeStruct((b, dm), jnp.float32),
          jax.ShapeDtypeStruct((b, dm), jnp.float32),
      ),
      mesh=mesh,
      scratch_types=(
          pltpu.VMEM((bpw,), jnp.int32),
          pltpu.VMEM((bpw,), jnp.int32),
          pltpu.VMEM((chunk, dm), jnp.float32),
          pltpu.VMEM((chunk, dm), jnp.float32),
          pltpu.SemaphoreType.DMA,
          pltpu.SemaphoreType.DMA,
          pltpu.SemaphoreType.DMA,
      ),
  )
  return fn(uid, iid, pw, s)


def _dot_body(gu, gi, out):
  out[...] = jnp.sum(gu[...] * gi[...], axis=1)


def _tc_dot(gu, gi):
  b = gu.shape[0]
  r = 4096
  dm = gu.shape[1]
  return pl.pallas_call(
      _dot_body,
      grid=(b // r,),
      in_specs=[
          pl.BlockSpec((r, dm), lambda i: (i, 0)),
          pl.BlockSpec((r, dm), lambda i: (i, 0)),
      ],
      out_specs=pl.BlockSpec((r,), lambda i: (i,)),
      out_shape=jax.ShapeDtypeStruct((b,), jnp.float32),
  )(gu, gi)


@jax.jit
def kernel(uid, iid, P, Q, new_feature, W, weu, wei):
  pt = P.T
  qt = Q.T
  nt = new_feature.T
  s = _make_s(qt, nt, W, wei)
  pw = _make_pw(pt, weu)
  gu, gi = _sc_gather(uid, iid, pw, s)
  return _tc_dot(gu, gi)


# speedup vs baseline: 2.7855x; 1.1111x over previous
import functools

import jax
import jax.numpy as jnp
from jax import lax
from jax.experimental import pallas as pl
from jax.experimental.pallas import tpu as pltpu
from jax.experimental.pallas import tpu_sc as plsc

NC = 2
NS = 16
NW = NC * NS


def _s_body(qt, nt, w, wei, out):
  dm = wei.shape[1]
  a = wei[0:dm, :]
  b2 = jnp.dot(w[...], wei[dm:2 * dm, :], preferred_element_type=jnp.float32)
  s = jnp.einsum("kb,kd->bd", qt[...], a,
                 preferred_element_type=jnp.float32)
  s = s + jnp.einsum("kb,kd->bd", nt[...], b2,
                     preferred_element_type=jnp.float32)
  out[...] = s


def _make_s(qt, nt, w, wei):
  isz = qt.shape[1]
  dq, dn = qt.shape[0], nt.shape[0]
  dm = wei.shape[1]
  blk = 16384
  grid = (isz + blk - 1) // blk
  return pl.pallas_call(
      _s_body,
      grid=(grid,),
      in_specs=[
          pl.BlockSpec((dq, blk), lambda i: (0, i)),
          pl.BlockSpec((dn, blk), lambda i: (0, i)),
          pl.BlockSpec((dn, dm), lambda i: (0, 0)),
          pl.BlockSpec((2 * dm, dm), lambda i: (0, 0)),
      ],
      out_specs=pl.BlockSpec((blk, dm), lambda i: (i, 0)),
      out_shape=jax.ShapeDtypeStruct((isz, dm), jnp.float32),
  )(qt, nt, w, wei)


def _pw_body(pt, weu, out):
  out[...] = jnp.einsum("kb,kd->bd", pt[...], weu[...],
                        preferred_element_type=jnp.float32)


def _make_pw(pt, weu):
  usz = pt.shape[1]
  dp = pt.shape[0]
  dm = weu.shape[1]
  blk = 16384
  grid = (usz + blk - 1) // blk
  return pl.pallas_call(
      _pw_body,
      grid=(grid,),
      in_specs=[
          pl.BlockSpec((dp, blk), lambda i: (0, i)),
          pl.BlockSpec((dp, dm), lambda i: (0, 0)),
      ],
      out_specs=pl.BlockSpec((blk, dm), lambda i: (i, 0)),
      out_shape=jax.ShapeDtypeStruct((usz, dm), jnp.float32),
  )(pt, weu)


def _gather_body(bpw, chunk, uid_h, iid_h, pw_h, s_h, outu_h, outi_h,
                 uid_v, iid_v, uv, iv, semi, semu, sems):
  wid = lax.axis_index("s") * NC + lax.axis_index("c")
  base = wid * bpw
  cu = pltpu.async_copy(uid_h.at[pl.ds(base, bpw)], uid_v, semi)
  ci = pltpu.async_copy(iid_h.at[pl.ds(base, bpw)], iid_v, semi)
  cu.wait()
  ci.wait()

  for h in range(bpw // chunk):
    off = h * chunk

    def issue(j, carry):
      uvec = uid_v[pl.ds(off + j * 16, 16)]
      ivec = iid_v[pl.ds(off + j * 16, 16)]
      for k in range(16):
        pltpu.async_copy(pw_h.at[uvec[k]], uv.at[j * 16 + k], semu)
        pltpu.async_copy(s_h.at[ivec[k]], iv.at[j * 16 + k], sems)
      return carry

    lax.fori_loop(0, chunk // 16, issue, 0)
    pltpu.make_async_copy(outu_h.at[pl.ds(0, chunk)], uv, semu).wait()
    pltpu.make_async_copy(outi_h.at[pl.ds(0, chunk)], iv, sems).wait()
    pltpu.sync_copy(uv, outu_h.at[pl.ds(base + off, chunk)])
    pltpu.sync_copy(iv, outi_h.at[pl.ds(base + off, chunk)])


def _sc_gather(uid, iid, pw, s):
  b = uid.shape[0]
  bpw = b // NW
  chunk = 256
  dm = pw.shape[1]
  mesh = plsc.VectorSubcoreMesh(core_axis_name="c", subcore_axis_name="s")
  fn = pl.kernel(
      functools.partial(_gather_body, bpw, chunk),
      out_type=(
          jax.ShapeDtypeStruct((b, dm), jnp.float32),
          jax.ShapeDtypeStruct((b, dm), jnp.float32),
      ),
      mesh=mesh,
      scratch_types=(
          pltpu.VMEM((bpw,), jnp.int32),
          pltpu.VMEM((bpw,), jnp.int32),
          pltpu.VMEM((chunk, dm), jnp.float32),
          pltpu.VMEM((chunk, dm), jnp.float32),
          pltpu.SemaphoreType.DMA,
          pltpu.SemaphoreType.DMA,
          pltpu.SemaphoreType.DMA,
      ),
  )
  return fn(uid, iid, pw, s)


def _dot_body(gu, gi, out):
  out[...] = jnp.sum(gu[...] * gi[...], axis=1)


def _tc_dot(gu, gi):
  b = gu.shape[0]
  r = 4096
  dm = gu.shape[1]
  return pl.pallas_call(
      _dot_body,
      grid=(b // r,),
      in_specs=[
          pl.BlockSpec((r, dm), lambda i: (i, 0)),
          pl.BlockSpec((r, dm), lambda i: (i, 0)),
      ],
      out_specs=pl.BlockSpec((r,), lambda i: (i,)),
      out_shape=jax.ShapeDtypeStruct((b,), jnp.float32),
  )(gu, gi)


@jax.jit
def kernel(uid, iid, P, Q, new_feature, W, weu, wei):
  pt = P.T
  qt = Q.T
  nt = new_feature.T
  s = _make_s(qt, nt, W, wei)
  pw = _make_pw(pt, weu)
  gu, gi = _sc_gather(uid, iid, pw, s)
  return _tc_dot(gu, gi)


# speedup vs baseline: 2.8974x; 1.0402x over previous
import functools

import jax
import jax.numpy as jnp
from jax import lax
from jax.experimental import pallas as pl
from jax.experimental.pallas import tpu as pltpu
from jax.experimental.pallas import tpu_sc as plsc

NC = 2
NS = 16
NW = NC * NS


def _s_body(qt, nt, w, wei, out):
  dm = wei.shape[1]
  a = wei[0:dm, :]
  b2 = jnp.dot(w[...], wei[dm:2 * dm, :], preferred_element_type=jnp.float32)
  s = jnp.einsum("kb,kd->bd", qt[...], a,
                 preferred_element_type=jnp.float32)
  s = s + jnp.einsum("kb,kd->bd", nt[...], b2,
                     preferred_element_type=jnp.float32)
  out[...] = s


def _make_s(qt, nt, w, wei):
  isz = qt.shape[1]
  dq, dn = qt.shape[0], nt.shape[0]
  dm = wei.shape[1]
  blk = 24576
  grid = (isz + blk - 1) // blk
  return pl.pallas_call(
      _s_body,
      grid=(grid,),
      in_specs=[
          pl.BlockSpec((dq, blk), lambda i: (0, i)),
          pl.BlockSpec((dn, blk), lambda i: (0, i)),
          pl.BlockSpec((dn, dm), lambda i: (0, 0)),
          pl.BlockSpec((2 * dm, dm), lambda i: (0, 0)),
      ],
      out_specs=pl.BlockSpec((blk, dm), lambda i: (i, 0)),
      out_shape=jax.ShapeDtypeStruct((isz, dm), jnp.float32),
  )(qt, nt, w, wei)


def _pw_body(pt, weu, out):
  out[...] = jnp.einsum("kb,kd->bd", pt[...], weu[...],
                        preferred_element_type=jnp.float32)


def _make_pw(pt, weu):
  usz = pt.shape[1]
  dp = pt.shape[0]
  dm = weu.shape[1]
  blk = 16384
  grid = (usz + blk - 1) // blk
  return pl.pallas_call(
      _pw_body,
      grid=(grid,),
      in_specs=[
          pl.BlockSpec((dp, blk), lambda i: (0, i)),
          pl.BlockSpec((dp, dm), lambda i: (0, 0)),
      ],
      out_specs=pl.BlockSpec((blk, dm), lambda i: (i, 0)),
      out_shape=jax.ShapeDtypeStruct((usz, dm), jnp.float32),
  )(pt, weu)


def _gather_body(bpw, chunk, uid_h, iid_h, pw_h, s_h, outu_h, outi_h,
                 uid_v, iid_v, uv, iv, semi, semu, sems):
  wid = lax.axis_index("s") * NC + lax.axis_index("c")
  base = wid * bpw
  cu = pltpu.async_copy(uid_h.at[pl.ds(base, bpw)], uid_v, semi)
  ci = pltpu.async_copy(iid_h.at[pl.ds(base, bpw)], iid_v, semi)
  cu.wait()
  ci.wait()

  for h in range(bpw // chunk):
    off = h * chunk

    def issue(j, carry):
      uvec = uid_v[pl.ds(off + j * 16, 16)]
      ivec = iid_v[pl.ds(off + j * 16, 16)]
      for k in range(16):
        pltpu.async_copy(pw_h.at[uvec[k]], uv.at[j * 16 + k], semu)
        pltpu.async_copy(s_h.at[ivec[k]], iv.at[j * 16 + k], sems)
      return carry

    lax.fori_loop(0, chunk // 16, issue, 0)
    pltpu.make_async_copy(outu_h.at[pl.ds(0, chunk)], uv, semu).wait()
    pltpu.make_async_copy(outi_h.at[pl.ds(0, chunk)], iv, sems).wait()
    pltpu.sync_copy(uv, outu_h.at[pl.ds(base + off, chunk)])
    pltpu.sync_copy(iv, outi_h.at[pl.ds(base + off, chunk)])


def _sc_gather(uid, iid, pw, s):
  b = uid.shape[0]
  bpw = b // NW
  chunk = 256
  dm = pw.shape[1]
  mesh = plsc.VectorSubcoreMesh(core_axis_name="c", subcore_axis_name="s")
  fn = pl.kernel(
      functools.partial(_gather_body, bpw, chunk),
      out_type=(
          jax.ShapeDtypeStruct((b, dm), jnp.float32),
          jax.ShapeDtypeStruct((b, dm), jnp.float32),
      ),
      mesh=mesh,
      scratch_types=(
          pltpu.VMEM((bpw,), jnp.int32),
          pltpu.VMEM((bpw,), jnp.int32),
          pltpu.VMEM((chunk, dm), jnp.float32),
          pltpu.VMEM((chunk, dm), jnp.float32),
          pltpu.SemaphoreType.DMA,
          pltpu.SemaphoreType.DMA,
          pltpu.SemaphoreType.DMA,
      ),
  )
  return fn(uid, iid, pw, s)


def _dot_body(gu, gi, out):
  out[...] = jnp.sum(gu[...] * gi[...], axis=1)


def _tc_dot(gu, gi):
  b = gu.shape[0]
  r = 4096
  dm = gu.shape[1]
  return pl.pallas_call(
      _dot_body,
      grid=(b // r,),
      in_specs=[
          pl.BlockSpec((r, dm), lambda i: (i, 0)),
          pl.BlockSpec((r, dm), lambda i: (i, 0)),
      ],
      out_specs=pl.BlockSpec((r,), lambda i: (i,)),
      out_shape=jax.ShapeDtypeStruct((b,), jnp.float32),
  )(gu, gi)


@jax.jit
def kernel(uid, iid, P, Q, new_feature, W, weu, wei):
  pt = P.T
  qt = Q.T
  nt = new_feature.T
  s = _make_s(qt, nt, W, wei)
  pw = _make_pw(pt, weu)
  gu, gi = _sc_gather(uid, iid, pw, s)
  return _tc_dot(gu, gi)
